# baseline (speedup 1.0000x reference)
import functools

import jax
import jax.numpy as jnp
from jax import lax
from jax.experimental import pallas as pl
from jax.experimental.pallas import tpu as pltpu
from jax.experimental.pallas import tpu_sc as plsc

B = 16384
D = 24
NT = 5
NC = 2
NS = 16
NW = NC * NS
BPW = B // NW

BV = 1024


def _tr_body(src, dst):
    dst[...] = jnp.transpose(src[...], (1, 0))


def _transpose_table(embT):
    d, v = embT.shape
    grid = (v + BV - 1) // BV
    return pl.pallas_call(
        _tr_body,
        grid=(grid,),
        in_specs=[pl.BlockSpec((d, BV), lambda i: (0, i))],
        out_specs=pl.BlockSpec((BV, d), lambda i: (i, 0)),
        out_shape=jax.ShapeDtypeStruct((v, d), jnp.float32),
    )(embT)


def _gather_body(i0, i1, i2, i3, i4, t0, t1, t2, t3, t4,
                 o0, o1, o2, o3, o4, idx_v, rows_v, sem):
    c = lax.axis_index("c")
    s = lax.axis_index("s")
    wid = s * NC + c
    base = wid * BPW
    for ih, th, oh in ((i0, t0, o0), (i1, t1, o1), (i2, t2, o2),
                       (i3, t3, o3), (i4, t4, o4)):
        pltpu.sync_copy(ih.at[pl.ds(base, BPW)], idx_v)

        def issue(k, _):
            v = idx_v[pl.ds(k * 16, 16)]
            for l in range(16):
                pltpu.async_copy(th.at[pl.ds(v[l], 1)],
                                 rows_v.at[pl.ds(k * 16 + l, 1)], sem)
            return 0

        lax.fori_loop(0, BPW // 16, issue, 0)
        pltpu.make_async_copy(th.at[pl.ds(0, BPW)], rows_v, sem).wait()
        pltpu.sync_copy(rows_v, oh.at[pl.ds(base, BPW)])


@jax.jit
def _sc_gather(i0, i1, i2, i3, i4, t0, t1, t2, t3, t4):
    mesh = plsc.VectorSubcoreMesh(core_axis_name="c", subcore_axis_name="s")
    f = functools.partial(
        pl.kernel,
        mesh=mesh,
        out_type=[jax.ShapeDtypeStruct((B, D), jnp.float32)] * NT,
        scratch_types=[
            pltpu.VMEM((BPW,), jnp.int32),
            pltpu.VMEM((BPW, D), jnp.float32),
            pltpu.SemaphoreType.DMA,
        ],
    )(_gather_body)
    return f(i0, i1, i2, i3, i4, t0, t1, t2, t3, t4)


BB = 2048
GRID = B // BB


def _mlp_body(g0, g1, g2, g3, g4, pts,
              we0, we1, we2, we3, we4, bemb,
              wnum, bnum, w1n, w1c, b1, w2, b2, out):
    dn = (((1,), (1,)), ((), ()))
    xc = lax.dot_general(g0[...], we0[...], dn,
                         preferred_element_type=jnp.float32)
    xc += lax.dot_general(g1[...], we1[...], dn,
                          preferred_element_type=jnp.float32)
    xc += lax.dot_general(g2[...], we2[...], dn,
                          preferred_element_type=jnp.float32)
    xc += lax.dot_general(g3[...], we3[...], dn,
                          preferred_element_type=jnp.float32)
    xc += lax.dot_general(g4[...], we4[...], dn,
                          preferred_element_type=jnp.float32)
    xc += bemb[...]
    xn = lax.dot_general(pts[...], wnum[...], dn,
                         preferred_element_type=jnp.float32) + bnum[...]
    h = lax.dot_general(xn, w1n[...], dn,
                        preferred_element_type=jnp.float32)
    h += lax.dot_general(xc, w1c[...], dn,
                         preferred_element_type=jnp.float32)
    h += b1[...]
    h = jnp.maximum(h, 0.0)
    out[...] = jnp.sum(h * w2[...], axis=1, keepdims=True) + b2[0, 0]


def _tc_mlp(gs, pts, wembs, bemb, wnum, bnum, w1n, w1c, b1, b2_w, b2):
    in_specs = (
        [pl.BlockSpec((BB, D), lambda i: (i, 0)) for _ in range(NT)]
        + [pl.BlockSpec((BB, 1), lambda i: (i, 0))]
        + [pl.BlockSpec((384, D), lambda i: (0, 0)) for _ in range(NT)]
        + [
            pl.BlockSpec((1, 384), lambda i: (0, 0)),
            pl.BlockSpec((128, 1), lambda i: (0, 0)),
            pl.BlockSpec((1, 128), lambda i: (0, 0)),
            pl.BlockSpec((256, 128), lambda i: (0, 0)),
            pl.BlockSpec((256, 384), lambda i: (0, 0)),
            pl.BlockSpec((1, 256), lambda i: (0, 0)),
            pl.BlockSpec((1, 256), lambda i: (0, 0)),
            pl.BlockSpec((1, 1), lambda i: (0, 0)),
        ]
    )
    return pl.pallas_call(
        _mlp_body,
        grid=(GRID,),
        in_specs=in_specs,
        out_specs=pl.BlockSpec((BB, 1), lambda i: (i, 0)),
        out_shape=jax.ShapeDtypeStruct((B, 1), jnp.float32),
    )(*gs, pts, *wembs, bemb, wnum, bnum, w1n, w1c, b1, b2_w, b2)


def kernel(country, province, region_1, variety, winery, points,
           emb_country, emb_province, emb_region_1, emb_variety, emb_winery,
           W_num, b_num, W_emb, b_emb, W_fc1, b_fc1, W_fc2, b_fc2):
    tabs = [_transpose_table(t.T) for t in
            (emb_country, emb_province, emb_region_1, emb_variety,
             emb_winery)]
    gs = _sc_gather(country, province, region_1, variety, winery, *tabs)
    wembs = [W_emb[:, t * D:(t + 1) * D] for t in range(NT)]
    w1n = W_fc1[:, :128]
    w1c = W_fc1[:, 128:]
    out = _tc_mlp(
        gs, points.reshape(B, 1), wembs, b_emb.reshape(1, 384),
        W_num, b_num.reshape(1, 128), w1n, w1c,
        b_fc1.reshape(1, 256), W_fc2, b_fc2.reshape(1, 1),
    )
    return out


# speedup vs baseline: 4.0778x; 4.0778x over previous
import functools

import jax
import jax.numpy as jnp
from jax import lax
from jax.experimental import pallas as pl
from jax.experimental.pallas import tpu as pltpu
from jax.experimental.pallas import tpu_sc as plsc

B = 16384
D = 24
NT = 5
NC = 2
NS = 16
NW = NC * NS
BPW = B // NW
CH = 16


def _winery_body(iw, tw, ow, idx_v, tiles, rows, sem):
    c = lax.axis_index("c")
    s = lax.axis_index("s")
    wid = s * NC + c
    base = wid * BPW
    pltpu.sync_copy(iw.at[pl.ds(base, BPW)], idx_v)
    iota = lax.iota(jnp.int32, 16)

    def chunk(g, _):
        v = idx_v[pl.ds(g * CH, CH)]
        cps = []
        for m in range(CH):
            off = pl.multiple_of(v[m] & -128, 128)
            cps.append(
                pltpu.async_copy(tw.at[:, pl.ds(off, 128)],
                                 tiles.at[pl.ds(m * D, D), :], sem))
        for cp in cps:
            cp.wait()
        for nn in range(CH * D // 16):
            rem = (16 * nn) % D
            q = (16 * nn) // D
            incr = 1 + ((iota + rem - D) >> 31)
            row = g * CH + q + incr
            rvec = plsc.load_gather(idx_v, [row])
            lane = jnp.bitwise_and(rvec, 127)
            val = plsc.load_gather(tiles, [nn * 16 + iota, lane])
            rows[pl.ds(g * (CH * D) + nn * 16, 16)] = val
        return 0

    lax.fori_loop(0, BPW // CH, chunk, 0)
    pltpu.sync_copy(rows, ow.at[pl.ds(base * D, BPW * D)])


@jax.jit
def _sc_gather_winery(iw, tw):
    mesh = plsc.VectorSubcoreMesh(core_axis_name="c", subcore_axis_name="s")
    f = functools.partial(
        pl.kernel,
        mesh=mesh,
        out_type=jax.ShapeDtypeStruct((B * D,), jnp.float32),
        scratch_types=[
            pltpu.VMEM((BPW,), jnp.int32),
            pltpu.VMEM((CH * D, 128), jnp.float32),
            pltpu.VMEM((BPW * D,), jnp.float32),
            pltpu.SemaphoreType.DMA,
        ],
        compiler_params=pltpu.CompilerParams(needs_layout_passes=False),
    )(_winery_body)
    return f(iw, tw)


def _gather_body(i0, i1, i2, i3, t0, t1, t2, t3,
                 o0, o1, o2, o3, idx_v, rows_v, sem):
    c = lax.axis_index("c")
    s = lax.axis_index("s")
    wid = s * NC + c
    base = wid * BPW
    for ih, th, oh in ((i0, t0, o0), (i1, t1, o1), (i2, t2, o2),
                       (i3, t3, o3)):
        pltpu.sync_copy(ih.at[pl.ds(base, BPW)], idx_v)

        def issue(k, _):
            v = idx_v[pl.ds(k * 16, 16)]
            for l in range(16):
                pltpu.async_copy(th.at[pl.ds(v[l], 1)],
                                 rows_v.at[pl.ds(k * 16 + l, 1)], sem)
            return 0

        lax.fori_loop(0, BPW // 16, issue, 0)
        pltpu.make_async_copy(th.at[pl.ds(0, BPW)], rows_v, sem).wait()
        pltpu.sync_copy(rows_v, oh.at[pl.ds(base, BPW)])


@jax.jit
def _sc_gather4(i0, i1, i2, i3, t0, t1, t2, t3):
    mesh = plsc.VectorSubcoreMesh(core_axis_name="c", subcore_axis_name="s")
    f = functools.partial(
        pl.kernel,
        mesh=mesh,
        out_type=[jax.ShapeDtypeStruct((B, D), jnp.float32)] * 4,
        scratch_types=[
            pltpu.VMEM((BPW,), jnp.int32),
            pltpu.VMEM((BPW, D), jnp.float32),
            pltpu.SemaphoreType.DMA,
        ],
    )(_gather_body)
    return f(i0, i1, i2, i3, t0, t1, t2, t3)


BB = 2048
GRID = B // BB


def _mlp_body(g0, g1, g2, g3, g4, pts,
              we0, we1, we2, we3, we4, bemb,
              wnum, bnum, w1n, w1c, b1, w2, b2, out):
    dn = (((1,), (1,)), ((), ()))
    xc = lax.dot_general(g0[...], we0[...], dn,
                         preferred_element_type=jnp.float32)
    xc += lax.dot_general(g1[...], we1[...], dn,
                          preferred_element_type=jnp.float32)
    xc += lax.dot_general(g2[...], we2[...], dn,
                          preferred_element_type=jnp.float32)
    xc += lax.dot_general(g3[...], we3[...], dn,
                          preferred_element_type=jnp.float32)
    xc += lax.dot_general(g4[...], we4[...], dn,
                          preferred_element_type=jnp.float32)
    xc += bemb[...]
    xn = lax.dot_general(pts[...], wnum[...], dn,
                         preferred_element_type=jnp.float32) + bnum[...]
    h = lax.dot_general(xn, w1n[...], dn,
                        preferred_element_type=jnp.float32)
    h += lax.dot_general(xc, w1c[...], dn,
                         preferred_element_type=jnp.float32)
    h += b1[...]
    h = jnp.maximum(h, 0.0)
    out[...] = jnp.sum(h * w2[...], axis=1, keepdims=True) + b2[0, 0]


def _tc_mlp(gs, pts, wembs, bemb, wnum, bnum, w1n, w1c, b1, b2_w, b2):
    in_specs = (
        [pl.BlockSpec((BB, D), lambda i: (i, 0)) for _ in range(NT)]
        + [pl.BlockSpec((BB, 1), lambda i: (i, 0))]
        + [pl.BlockSpec((384, D), lambda i: (0, 0)) for _ in range(NT)]
        + [
            pl.BlockSpec((1, 384), lambda i: (0, 0)),
            pl.BlockSpec((128, 1), lambda i: (0, 0)),
            pl.BlockSpec((1, 128), lambda i: (0, 0)),
            pl.BlockSpec((256, 128), lambda i: (0, 0)),
            pl.BlockSpec((256, 384), lambda i: (0, 0)),
            pl.BlockSpec((1, 256), lambda i: (0, 0)),
            pl.BlockSpec((1, 256), lambda i: (0, 0)),
            pl.BlockSpec((1, 1), lambda i: (0, 0)),
        ]
    )
    return pl.pallas_call(
        _mlp_body,
        grid=(GRID,),
        in_specs=in_specs,
        out_specs=pl.BlockSpec((BB, 1), lambda i: (i, 0)),
        out_shape=jax.ShapeDtypeStruct((B, 1), jnp.float32),
    )(*gs, pts, *wembs, bemb, wnum, bnum, w1n, w1c, b1, b2_w, b2)


def kernel(country, province, region_1, variety, winery, points,
           emb_country, emb_province, emb_region_1, emb_variety, emb_winery,
           W_num, b_num, W_emb, b_emb, W_fc1, b_fc1, W_fc2, b_fc2):
    g4 = _sc_gather_winery(winery, emb_winery.T).reshape(B, D)
    gs = _sc_gather4(country, province, region_1, variety,
                     emb_country, emb_province, emb_region_1, emb_variety)
    wembs = [W_emb[:, t * D:(t + 1) * D] for t in range(NT)]
    w1n = W_fc1[:, :128]
    w1c = W_fc1[:, 128:]
    out = _tc_mlp(
        list(gs) + [g4], points.reshape(B, 1), wembs, b_emb.reshape(1, 384),
        W_num, b_num.reshape(1, 128), w1n, w1c,
        b_fc1.reshape(1, 256), W_fc2, b_fc2.reshape(1, 1),
    )
    return out


# speedup vs baseline: 4.1551x; 1.0190x over previous
import functools

import jax
import jax.numpy as jnp
from jax import lax
from jax.experimental import pallas as pl
from jax.experimental.pallas import tpu as pltpu
from jax.experimental.pallas import tpu_sc as plsc

B = 16384
D = 24
NT = 5
NC = 2
NS = 16
NW = NC * NS
BPW = B // NW
CH = 16


def _winery_body(iw, tw, ow, idx_v, tiles, rows, sem):
    c = lax.axis_index("c")
    s = lax.axis_index("s")
    wid = s * NC + c
    base = wid * BPW
    pltpu.sync_copy(iw.at[pl.ds(base, BPW)], idx_v)
    iota = lax.iota(jnp.int32, 16)

    def chunk(g, _):
        v = idx_v[pl.ds(g * CH, CH)]
        cps = []
        for m in range(CH):
            off = pl.multiple_of(v[m] & -128, 128)
            cps.append(
                pltpu.async_copy(tw.at[:, pl.ds(off, 128)],
                                 tiles.at[pl.ds(m * D, D), :], sem))
        for cp in cps:
            cp.wait()
        for nn in range(CH * D // 16):
            rem = (16 * nn) % D
            q = (16 * nn) // D
            incr = 1 + ((iota + rem - D) >> 31)
            row = g * CH + q + incr
            rvec = plsc.load_gather(idx_v, [row])
            lane = jnp.bitwise_and(rvec, 127)
            val = plsc.load_gather(tiles, [nn * 16 + iota, lane])
            rows[pl.ds(g * (CH * D) + nn * 16, 16)] = val
        return 0

    lax.fori_loop(0, BPW // CH, chunk, 0)
    pltpu.sync_copy(rows, ow.at[pl.ds(base * D, BPW * D)])


@jax.jit
def _sc_gather_winery(iw, tw):
    mesh = plsc.VectorSubcoreMesh(core_axis_name="c", subcore_axis_name="s")
    f = functools.partial(
        pl.kernel,
        mesh=mesh,
        out_type=jax.ShapeDtypeStruct((B * D,), jnp.float32),
        scratch_types=[
            pltpu.VMEM((BPW,), jnp.int32),
            pltpu.VMEM((CH * D, 128), jnp.float32),
            pltpu.VMEM((BPW * D,), jnp.float32),
            pltpu.SemaphoreType.DMA,
        ],
        compiler_params=pltpu.CompilerParams(needs_layout_passes=False),
    )(_winery_body)
    return f(iw, tw)


def _gather_body(i0, i1, i2, i3, t0, t1, t2, t3,
                 o0, o1, o2, o3, idx_v, rows_v, sem):
    c = lax.axis_index("c")
    s = lax.axis_index("s")
    wid = s * NC + c
    base = wid * BPW
    for ih, th, oh in ((i0, t0, o0), (i1, t1, o1), (i2, t2, o2),
                       (i3, t3, o3)):
        pltpu.sync_copy(ih.at[pl.ds(base, BPW)], idx_v)

        def issue(k, _):
            v = idx_v[pl.ds(k * 16, 16)]
            for l in range(16):
                pltpu.async_copy(th.at[pl.ds(v[l], 1)],
                                 rows_v.at[pl.ds(k * 16 + l, 1)], sem)
            return 0

        lax.fori_loop(0, BPW // 16, issue, 0)
        pltpu.make_async_copy(th.at[pl.ds(0, BPW)], rows_v, sem).wait()
        pltpu.sync_copy(rows_v, oh.at[pl.ds(base, BPW)])


@jax.jit
def _sc_gather4(i0, i1, i2, i3, t0, t1, t2, t3):
    mesh = plsc.VectorSubcoreMesh(core_axis_name="c", subcore_axis_name="s")
    f = functools.partial(
        pl.kernel,
        mesh=mesh,
        out_type=[jax.ShapeDtypeStruct((B, D), jnp.float32)] * 4,
        scratch_types=[
            pltpu.VMEM((BPW,), jnp.int32),
            pltpu.VMEM((BPW, D), jnp.float32),
            pltpu.SemaphoreType.DMA,
        ],
    )(_gather_body)
    return f(i0, i1, i2, i3, t0, t1, t2, t3)


BB = 2048
GRID = B // BB


def _mlp_body(g0, g1, g2, g3, g4, pts,
              we0, we1, we2, we3, we4, bemb,
              wnum, bnum, w1n, w1c, b1, w2, b2, out):
    dt = (((1,), (1,)), ((), ()))
    d10 = (((1,), (0,)), ((), ()))
    cst = lax.dot_general(bemb[...], w1c[...], dt,
                          preferred_element_type=jnp.float32)
    cst += lax.dot_general(bnum[...], w1n[...], dt,
                           preferred_element_type=jnp.float32)
    cst += b1[...]
    u = jnp.sum(w1n[...] * wnum[...], axis=1, keepdims=True)
    acc = lax.dot_general(pts[...], u, dt,
                          preferred_element_type=jnp.float32)
    for g, we in ((g0, we0), (g1, we1), (g2, we2), (g3, we3), (g4, we4)):
        m = lax.dot_general(w1c[...], we[...], d10,
                            preferred_element_type=jnp.float32)
        acc += lax.dot_general(g[...], m, dt,
                               preferred_element_type=jnp.float32)
    h = jnp.maximum(acc + cst, 0.0)
    out[...] = jnp.sum(h * w2[...], axis=1, keepdims=True) + b2[0, 0]


def _tc_mlp(gs, pts, wembs, bemb, wnum, bnum, w1n, w1c, b1, b2_w, b2):
    in_specs = (
        [pl.BlockSpec((BB, D), lambda i: (i, 0)) for _ in range(NT)]
        + [pl.BlockSpec((BB, 1), lambda i: (i, 0))]
        + [pl.BlockSpec((384, D), lambda i: (0, 0)) for _ in range(NT)]
        + [
            pl.BlockSpec((1, 384), lambda i: (0, 0)),
            pl.BlockSpec((1, 128), lambda i: (0, 0)),
            pl.BlockSpec((1, 128), lambda i: (0, 0)),
            pl.BlockSpec((256, 128), lambda i: (0, 0)),
            pl.BlockSpec((256, 384), lambda i: (0, 0)),
            pl.BlockSpec((1, 256), lambda i: (0, 0)),
            pl.BlockSpec((1, 256), lambda i: (0, 0)),
            pl.BlockSpec((1, 1), lambda i: (0, 0)),
        ]
    )
    return pl.pallas_call(
        _mlp_body,
        grid=(GRID,),
        in_specs=in_specs,
        out_specs=pl.BlockSpec((BB, 1), lambda i: (i, 0)),
        out_shape=jax.ShapeDtypeStruct((B, 1), jnp.float32),
    )(*gs, pts, *wembs, bemb, wnum, bnum, w1n, w1c, b1, b2_w, b2)


def kernel(country, province, region_1, variety, winery, points,
           emb_country, emb_province, emb_region_1, emb_variety, emb_winery,
           W_num, b_num, W_emb, b_emb, W_fc1, b_fc1, W_fc2, b_fc2):
    g4 = _sc_gather_winery(winery, emb_winery.T).reshape(B, D)
    gs = _sc_gather4(country, province, region_1, variety,
                     emb_country, emb_province, emb_region_1, emb_variety)
    wembs = [W_emb[:, t * D:(t + 1) * D] for t in range(NT)]
    w1n = W_fc1[:, :128]
    w1c = W_fc1[:, 128:]
    out = _tc_mlp(
        list(gs) + [g4], points.reshape(B, 1), wembs, b_emb.reshape(1, 384),
        W_num.reshape(1, 128), b_num.reshape(1, 128), w1n, w1c,
        b_fc1.reshape(1, 256), W_fc2, b_fc2.reshape(1, 1),
    )
    return out


# speedup vs baseline: 4.1944x; 1.0095x over previous
import functools

import jax
import jax.numpy as jnp
from jax import lax
from jax.experimental import pallas as pl
from jax.experimental.pallas import tpu as pltpu
from jax.experimental.pallas import tpu_sc as plsc

B = 16384
D = 24
NT = 5
NC = 2
NS = 16
NW = NC * NS
BPW = B // NW
CH = 16


def _winery_body(iw, tw, ow, idx_v, tiles0, tiles1, rows, sem0, sem1):
    c = lax.axis_index("c")
    s = lax.axis_index("s")
    wid = s * NC + c
    base = wid * BPW
    pltpu.sync_copy(iw.at[pl.ds(base, BPW)], idx_v)
    iota = lax.iota(jnp.int32, 16)
    NCH = BPW // CH

    def issue(g, tiles, sem):
        v = idx_v[pl.ds(g * CH, CH)]
        for m in range(CH):
            off = pl.multiple_of(v[m] & -128, 128)
            pltpu.async_copy(tw.at[:, pl.ds(off, 128)],
                             tiles.at[pl.ds(m * D, D), :], sem)

    def drain(tiles, sem):
        for m in range(CH):
            pltpu.make_async_copy(tw.at[:, pl.ds(0, 128)],
                                  tiles.at[pl.ds(m * D, D), :], sem).wait()

    def extract(g, tiles):
        for nn in range(CH * D // 16):
            rem = (16 * nn) % D
            q = (16 * nn) // D
            incr = 1 + ((iota + rem - D) >> 31)
            row = g * CH + q + incr
            rvec = plsc.load_gather(idx_v, [row])
            lane = jnp.bitwise_and(rvec, 127)
            val = plsc.load_gather(tiles, [nn * 16 + iota, lane])
            rows[pl.ds(g * (CH * D) + nn * 16, 16)] = val

    issue(0, tiles0, sem0)

    def pair(p, _):
        issue(2 * p + 1, tiles1, sem1)
        drain(tiles0, sem0)
        extract(2 * p, tiles0)
        issue((2 * p + 2) % NCH, tiles0, sem0)
        drain(tiles1, sem1)
        extract(2 * p + 1, tiles1)
        return 0

    lax.fori_loop(0, NCH // 2, pair, 0)
    drain(tiles0, sem0)
    pltpu.sync_copy(rows, ow.at[pl.ds(base * D, BPW * D)])


@jax.jit
def _sc_gather_winery(iw, tw):
    mesh = plsc.VectorSubcoreMesh(core_axis_name="c", subcore_axis_name="s")
    f = functools.partial(
        pl.kernel,
        mesh=mesh,
        out_type=jax.ShapeDtypeStruct((B * D,), jnp.float32),
        scratch_types=[
            pltpu.VMEM((BPW,), jnp.int32),
            pltpu.VMEM((CH * D, 128), jnp.float32),
            pltpu.VMEM((CH * D, 128), jnp.float32),
            pltpu.VMEM((BPW * D,), jnp.float32),
            pltpu.SemaphoreType.DMA,
            pltpu.SemaphoreType.DMA,
        ],
        compiler_params=pltpu.CompilerParams(needs_layout_passes=False),
    )(_winery_body)
    return f(iw, tw)


def _gather_body(i0, i1, i2, i3, t0, t1, t2, t3,
                 o0, o1, o2, o3, idx_v, rows_v, sem):
    c = lax.axis_index("c")
    s = lax.axis_index("s")
    wid = s * NC + c
    base = wid * BPW
    for ih, th, oh in ((i0, t0, o0), (i1, t1, o1), (i2, t2, o2),
                       (i3, t3, o3)):
        pltpu.sync_copy(ih.at[pl.ds(base, BPW)], idx_v)

        def issue(k, _):
            v = idx_v[pl.ds(k * 16, 16)]
            for l in range(16):
                pltpu.async_copy(th.at[pl.ds(v[l], 1)],
                                 rows_v.at[pl.ds(k * 16 + l, 1)], sem)
            return 0

        lax.fori_loop(0, BPW // 16, issue, 0)
        pltpu.make_async_copy(th.at[pl.ds(0, BPW)], rows_v, sem).wait()
        pltpu.sync_copy(rows_v, oh.at[pl.ds(base, BPW)])


@jax.jit
def _sc_gather4(i0, i1, i2, i3, t0, t1, t2, t3):
    mesh = plsc.VectorSubcoreMesh(core_axis_name="c", subcore_axis_name="s")
    f = functools.partial(
        pl.kernel,
        mesh=mesh,
        out_type=[jax.ShapeDtypeStruct((B, D), jnp.float32)] * 4,
        scratch_types=[
            pltpu.VMEM((BPW,), jnp.int32),
            pltpu.VMEM((BPW, D), jnp.float32),
            pltpu.SemaphoreType.DMA,
        ],
    )(_gather_body)
    return f(i0, i1, i2, i3, t0, t1, t2, t3)


BB = 2048
GRID = B // BB


def _mlp_body(g0, g1, g2, g3, g4, pts,
              we0, we1, we2, we3, we4, bemb,
              wnum, bnum, w1n, w1c, b1, w2, b2, out):
    dt = (((1,), (1,)), ((), ()))
    d10 = (((1,), (0,)), ((), ()))
    cst = lax.dot_general(bemb[...], w1c[...], dt,
                          preferred_element_type=jnp.float32)
    cst += lax.dot_general(bnum[...], w1n[...], dt,
                           preferred_element_type=jnp.float32)
    cst += b1[...]
    u = jnp.sum(w1n[...] * wnum[...], axis=1, keepdims=True)
    acc = lax.dot_general(pts[...], u, dt,
                          preferred_element_type=jnp.float32)
    for g, we in ((g0, we0), (g1, we1), (g2, we2), (g3, we3), (g4, we4)):
        m = lax.dot_general(w1c[...], we[...], d10,
                            preferred_element_type=jnp.float32)
        acc += lax.dot_general(g[...], m, dt,
                               preferred_element_type=jnp.float32)
    h = jnp.maximum(acc + cst, 0.0)
    out[...] = jnp.sum(h * w2[...], axis=1, keepdims=True) + b2[0, 0]


def _tc_mlp(gs, pts, wembs, bemb, wnum, bnum, w1n, w1c, b1, b2_w, b2):
    in_specs = (
        [pl.BlockSpec((BB, D), lambda i: (i, 0)) for _ in range(NT)]
        + [pl.BlockSpec((BB, 1), lambda i: (i, 0))]
        + [pl.BlockSpec((384, D), lambda i: (0, 0)) for _ in range(NT)]
        + [
            pl.BlockSpec((1, 384), lambda i: (0, 0)),
            pl.BlockSpec((1, 128), lambda i: (0, 0)),
            pl.BlockSpec((1, 128), lambda i: (0, 0)),
            pl.BlockSpec((256, 128), lambda i: (0, 0)),
            pl.BlockSpec((256, 384), lambda i: (0, 0)),
            pl.BlockSpec((1, 256), lambda i: (0, 0)),
            pl.BlockSpec((1, 256), lambda i: (0, 0)),
            pl.BlockSpec((1, 1), lambda i: (0, 0)),
        ]
    )
    return pl.pallas_call(
        _mlp_body,
        grid=(GRID,),
        in_specs=in_specs,
        out_specs=pl.BlockSpec((BB, 1), lambda i: (i, 0)),
        out_shape=jax.ShapeDtypeStruct((B, 1), jnp.float32),
    )(*gs, pts, *wembs, bemb, wnum, bnum, w1n, w1c, b1, b2_w, b2)


def kernel(country, province, region_1, variety, winery, points,
           emb_country, emb_province, emb_region_1, emb_variety, emb_winery,
           W_num, b_num, W_emb, b_emb, W_fc1, b_fc1, W_fc2, b_fc2):
    g4 = _sc_gather_winery(winery, emb_winery.T).reshape(B, D)
    gs = _sc_gather4(country, province, region_1, variety,
                     emb_country, emb_province, emb_region_1, emb_variety)
    wembs = [W_emb[:, t * D:(t + 1) * D] for t in range(NT)]
    w1n = W_fc1[:, :128]
    w1c = W_fc1[:, 128:]
    out = _tc_mlp(
        list(gs) + [g4], points.reshape(B, 1), wembs, b_emb.reshape(1, 384),
        W_num.reshape(1, 128), b_num.reshape(1, 128), w1n, w1c,
        b_fc1.reshape(1, 256), W_fc2, b_fc2.reshape(1, 1),
    )
    return out


# speedup vs baseline: 4.1955x; 1.0002x over previous
import functools

import jax
import jax.numpy as jnp
from jax import lax
from jax.experimental import pallas as pl
from jax.experimental.pallas import tpu as pltpu
from jax.experimental.pallas import tpu_sc as plsc

B = 16384
D = 24
NT = 5
NC = 2
NS = 16
NW = NC * NS
BPW = B // NW
CH = 16


def _winery_body(iw, tw, ow, idx_v, tiles0, tiles1, rows, sem0, sem1):
    c = lax.axis_index("c")
    s = lax.axis_index("s")
    wid = s * NC + c
    base = wid * BPW
    pltpu.sync_copy(iw.at[pl.ds(base, BPW)], idx_v)
    iota = lax.iota(jnp.int32, 16)
    NCH = BPW // CH

    def issue(g, tiles, sem):
        v = idx_v[pl.ds(g * CH, CH)]
        for m in range(CH):
            off = pl.multiple_of(v[m] & -128, 128)
            pltpu.async_copy(tw.at[:, pl.ds(off, 128)],
                             tiles.at[pl.ds(m * D, D), :], sem)

    def drain(tiles, sem):
        for m in range(CH):
            pltpu.make_async_copy(tw.at[:, pl.ds(0, 128)],
                                  tiles.at[pl.ds(m * D, D), :], sem).wait()

    def extract(g, tiles):
        for nn in range(CH * D // 16):
            rem = (16 * nn) % D
            q = (16 * nn) // D
            incr = 1 + ((iota + rem - D) >> 31)
            row = g * CH + q + incr
            rvec = plsc.load_gather(idx_v, [row])
            lane = jnp.bitwise_and(rvec, 127)
            val = plsc.load_gather(tiles, [nn * 16 + iota, lane])
            rows[pl.ds(g * (CH * D) + nn * 16, 16)] = val

    issue(0, tiles0, sem0)

    def pair(p, _):
        issue(2 * p + 1, tiles1, sem1)
        drain(tiles0, sem0)
        extract(2 * p, tiles0)
        issue((2 * p + 2) % NCH, tiles0, sem0)
        drain(tiles1, sem1)
        extract(2 * p + 1, tiles1)
        return 0

    lax.fori_loop(0, NCH // 2, pair, 0)
    drain(tiles0, sem0)
    pltpu.sync_copy(rows, ow.at[pl.ds(base * D, BPW * D)])


@jax.jit
def _sc_gather_winery(iw, tw):
    mesh = plsc.VectorSubcoreMesh(core_axis_name="c", subcore_axis_name="s")
    f = functools.partial(
        pl.kernel,
        mesh=mesh,
        out_type=jax.ShapeDtypeStruct((B * D,), jnp.float32),
        scratch_types=[
            pltpu.VMEM((BPW,), jnp.int32),
            pltpu.VMEM((CH * D, 128), jnp.float32),
            pltpu.VMEM((CH * D, 128), jnp.float32),
            pltpu.VMEM((BPW * D,), jnp.float32),
            pltpu.SemaphoreType.DMA,
            pltpu.SemaphoreType.DMA,
        ],
        compiler_params=pltpu.CompilerParams(needs_layout_passes=False),
    )(_winery_body)
    return f(iw, tw)


def _gather_body(i0, i1, i2, i3, t0, t1, t2, t3,
                 o0, o1, o2, o3, idx_v, rows_v, sem):
    c = lax.axis_index("c")
    s = lax.axis_index("s")
    wid = s * NC + c
    base = wid * BPW
    for ih, th, oh in ((i0, t0, o0), (i1, t1, o1), (i2, t2, o2),
                       (i3, t3, o3)):
        pltpu.sync_copy(ih.at[pl.ds(base, BPW)], idx_v)

        def issue(k, _):
            v = idx_v[pl.ds(k * 16, 16)]
            for l in range(16):
                pltpu.async_copy(th.at[pl.ds(v[l], 1)],
                                 rows_v.at[pl.ds(k * 16 + l, 1)], sem)
            return 0

        lax.fori_loop(0, BPW // 16, issue, 0)
        pltpu.make_async_copy(th.at[pl.ds(0, BPW)], rows_v, sem).wait()
        pltpu.sync_copy(rows_v, oh.at[pl.ds(base, BPW)])


@jax.jit
def _sc_gather4(i0, i1, i2, i3, t0, t1, t2, t3):
    mesh = plsc.VectorSubcoreMesh(core_axis_name="c", subcore_axis_name="s")
    f = functools.partial(
        pl.kernel,
        mesh=mesh,
        out_type=[jax.ShapeDtypeStruct((B, D), jnp.float32)] * 4,
        scratch_types=[
            pltpu.VMEM((BPW,), jnp.int32),
            pltpu.VMEM((BPW, D), jnp.float32),
            pltpu.SemaphoreType.DMA,
        ],
    )(_gather_body)
    return f(i0, i1, i2, i3, t0, t1, t2, t3)


BB = 4096
GRID = B // BB


def _mlp_body(g0, g1, g2, g3, g4, pts,
              we0, we1, we2, we3, we4, bemb,
              wnum, bnum, w1n, w1c, b1, w2, b2, out):
    dt = (((1,), (1,)), ((), ()))
    d10 = (((1,), (0,)), ((), ()))
    cst = lax.dot_general(bemb[...], w1c[...], dt,
                          preferred_element_type=jnp.float32)
    cst += lax.dot_general(bnum[...], w1n[...], dt,
                           preferred_element_type=jnp.float32)
    cst += b1[...]
    u = jnp.sum(w1n[...] * wnum[...], axis=1, keepdims=True)
    acc = lax.dot_general(pts[...], u, dt,
                          preferred_element_type=jnp.float32)
    for g, we in ((g0, we0), (g1, we1), (g2, we2), (g3, we3), (g4, we4)):
        m = lax.dot_general(w1c[...], we[...], d10,
                            preferred_element_type=jnp.float32)
        acc += lax.dot_general(g[...], m, dt,
                               preferred_element_type=jnp.float32)
    h = jnp.maximum(acc + cst, 0.0)
    out[...] = jnp.sum(h * w2[...], axis=1, keepdims=True) + b2[0, 0]


def _tc_mlp(gs, pts, wembs, bemb, wnum, bnum, w1n, w1c, b1, b2_w, b2):
    in_specs = (
        [pl.BlockSpec((BB, D), lambda i: (i, 0)) for _ in range(NT)]
        + [pl.BlockSpec((BB, 1), lambda i: (i, 0))]
        + [pl.BlockSpec((384, D), lambda i: (0, 0)) for _ in range(NT)]
        + [
            pl.BlockSpec((1, 384), lambda i: (0, 0)),
            pl.BlockSpec((1, 128), lambda i: (0, 0)),
            pl.BlockSpec((1, 128), lambda i: (0, 0)),
            pl.BlockSpec((256, 128), lambda i: (0, 0)),
            pl.BlockSpec((256, 384), lambda i: (0, 0)),
            pl.BlockSpec((1, 256), lambda i: (0, 0)),
            pl.BlockSpec((1, 256), lambda i: (0, 0)),
            pl.BlockSpec((1, 1), lambda i: (0, 0)),
        ]
    )
    return pl.pallas_call(
        _mlp_body,
        grid=(GRID,),
        in_specs=in_specs,
        out_specs=pl.BlockSpec((BB, 1), lambda i: (i, 0)),
        out_shape=jax.ShapeDtypeStruct((B, 1), jnp.float32),
    )(*gs, pts, *wembs, bemb, wnum, bnum, w1n, w1c, b1, b2_w, b2)


def kernel(country, province, region_1, variety, winery, points,
           emb_country, emb_province, emb_region_1, emb_variety, emb_winery,
           W_num, b_num, W_emb, b_emb, W_fc1, b_fc1, W_fc2, b_fc2):
    g4 = _sc_gather_winery(winery, emb_winery.T).reshape(B, D)
    gs = _sc_gather4(country, province, region_1, variety,
                     emb_country, emb_province, emb_region_1, emb_variety)
    wembs = [W_emb[:, t * D:(t + 1) * D] for t in range(NT)]
    w1n = W_fc1[:, :128]
    w1c = W_fc1[:, 128:]
    out = _tc_mlp(
        list(gs) + [g4], points.reshape(B, 1), wembs, b_emb.reshape(1, 384),
        W_num.reshape(1, 128), b_num.reshape(1, 128), w1n, w1c,
        b_fc1.reshape(1, 256), W_fc2, b_fc2.reshape(1, 1),
    )
    return out
